# Initial kernel scaffold; baseline (speedup 1.0000x reference)
import jax
import jax.numpy as jnp
from jax.experimental import pallas as pl


def kernel(pretrained, finetuned):
    raise NotImplementedError("write your pallas kernel here")


# speedup vs baseline: 84.8234x; 84.8234x over previous
import functools

import jax
import jax.numpy as jnp
from jax import lax
from jax.experimental import pallas as pl
from jax.experimental.pallas import tpu as pltpu
from jax.experimental.pallas import tpu_sc as plsc

_N = 16777216
_K = max(1, int(0.005 * _N))
_BIAS = 5.0

_NC = 2
_NS = 16
_NW = _NC * _NS
_PER_W = _N // _NW
_L = 16

_NB1 = 32768
_NB2 = 65536
_CH1 = 16384
_CH2 = 8192
_NBUF = 2


def _zero_hist2d(hist, rows):
    zeros = jnp.zeros((_L,), jnp.int32)

    def zbody(i, carry):
        hist[i >> 3, pl.ds((i & 7) * _L, _L)] = zeros
        return carry

    lax.fori_loop(0, rows * 8, zbody, 0, unroll=8)


def _hist_pass(p_hbm, f_hbm, pbufs, fbufs, sems, base, chunk, update):
    nchunk = _PER_W // chunk
    for b in range(_NBUF):
        off = base + b * chunk
        pltpu.async_copy(p_hbm.at[pl.ds(off, chunk)], pbufs[b], sems[b])
        pltpu.async_copy(f_hbm.at[pl.ds(off, chunk)], fbufs[b], sems[b])

    def chunk_body(g, carry):
        for b in range(_NBUF):
            ci = g * _NBUF + b
            pltpu.make_async_copy(p_hbm.at[pl.ds(0, chunk)], pbufs[b],
                                  sems[b]).wait()
            pltpu.make_async_copy(f_hbm.at[pl.ds(0, chunk)], fbufs[b],
                                  sems[b]).wait()

            @plsc.parallel_loop(0, chunk // _L, unroll=8)
            def vec_body(iv, _b=b):
                pv = pbufs[_b][pl.ds(iv * _L, _L)]
                fv = fbufs[_b][pl.ds(iv * _L, _L)]
                update(pv, fv)

            @pl.when(ci + _NBUF < nchunk)
            def _(b=b, ci=ci):
                off = base + (ci + _NBUF) * chunk
                pltpu.async_copy(p_hbm.at[pl.ds(off, chunk)], pbufs[b],
                                 sems[b])
                pltpu.async_copy(f_hbm.at[pl.ds(off, chunk)], fbufs[b],
                                 sems[b])
        return carry

    lax.fori_loop(0, nchunk // _NBUF, chunk_body, 0)


def _sc_hist_hi_body(p_hbm, f_hbm, out_hbm, pb0, pb1, fb0, fb1, hist,
                     sem0, sem1):
    c = lax.axis_index("c")
    s = lax.axis_index("s")
    wid = s * _NC + c
    ones = jnp.ones((_L,), jnp.int32)
    sh16 = jnp.full((_L,), 16, jnp.int32)
    sh7 = jnp.full((_L,), 7, jnp.int32)
    m127 = jnp.full((_L,), 127, jnp.int32)
    _zero_hist2d(hist, _NB1 // 128)

    def update(pv, fv):
        bits = lax.bitcast_convert_type(jnp.abs(fv - pv), jnp.int32)
        hi = lax.shift_right_logical(bits, sh16)
        plsc.addupdate_scatter(
            hist, [lax.shift_right_logical(hi, sh7),
                   jnp.bitwise_and(hi, m127)], ones)

    _hist_pass(p_hbm, f_hbm, (pb0, pb1), (fb0, fb1), (sem0, sem1),
               wid * _PER_W, _CH1, update)
    pltpu.sync_copy(hist, out_hbm.at[wid])


@functools.cache
def _sc_hist_hi():
    return pl.kernel(
        _sc_hist_hi_body,
        out_type=jax.ShapeDtypeStruct((_NW, _NB1 // 128, 128), jnp.int32),
        mesh=plsc.VectorSubcoreMesh(core_axis_name="c", subcore_axis_name="s",
                                    num_cores=_NC, num_subcores=_NS),
        compiler_params=pltpu.CompilerParams(needs_layout_passes=False),
        scratch_types=[
            pltpu.VMEM((_CH1,), jnp.float32),
            pltpu.VMEM((_CH1,), jnp.float32),
            pltpu.VMEM((_CH1,), jnp.float32),
            pltpu.VMEM((_CH1,), jnp.float32),
            pltpu.VMEM((_NB1 // 128, 128), jnp.int32),
            pltpu.SemaphoreType.DMA,
            pltpu.SemaphoreType.DMA,
        ],
    )


def _sc_hist_lo_body(p_hbm, f_hbm, b1_hbm, out_hbm, pb0, pb1, fb0, fb1,
                     bbuf, hist, sem0, sem1):
    c = lax.axis_index("c")
    s = lax.axis_index("s")
    wid = s * _NC + c
    ones = jnp.ones((_L,), jnp.int32)
    sh16 = jnp.full((_L,), 16, jnp.int32)
    sh7 = jnp.full((_L,), 7, jnp.int32)
    m127 = jnp.full((_L,), 127, jnp.int32)
    lomask = jnp.full((_L,), 0xFFFF, jnp.int32)
    _zero_hist2d(hist, _NB2 // 128)
    pltpu.sync_copy(b1_hbm, bbuf)
    b1v = bbuf[...]

    def update(pv, fv):
        bits = lax.bitcast_convert_type(jnp.abs(fv - pv), jnp.int32)
        hi = lax.shift_right_logical(bits, sh16)
        lo = jnp.bitwise_and(bits, lomask)
        plsc.addupdate_scatter(
            hist, [lax.shift_right_logical(lo, sh7),
                   jnp.bitwise_and(lo, m127)], ones, mask=hi == b1v)

    _hist_pass(p_hbm, f_hbm, (pb0, pb1), (fb0, fb1), (sem0, sem1),
               wid * _PER_W, _CH2, update)
    pltpu.sync_copy(hist, out_hbm.at[wid])


@functools.cache
def _sc_hist_lo():
    return pl.kernel(
        _sc_hist_lo_body,
        out_type=jax.ShapeDtypeStruct((_NW, _NB2 // 128, 128), jnp.int32),
        mesh=plsc.VectorSubcoreMesh(core_axis_name="c", subcore_axis_name="s",
                                    num_cores=_NC, num_subcores=_NS),
        compiler_params=pltpu.CompilerParams(needs_layout_passes=False),
        scratch_types=[
            pltpu.VMEM((_CH2,), jnp.float32),
            pltpu.VMEM((_CH2,), jnp.float32),
            pltpu.VMEM((_CH2,), jnp.float32),
            pltpu.VMEM((_CH2,), jnp.float32),
            pltpu.VMEM((_L,), jnp.int32),
            pltpu.VMEM((_NB2 // 128, 128), jnp.int32),
            pltpu.SemaphoreType.DMA,
            pltpu.SemaphoreType.DMA,
        ],
    )


def _flat_suffix(h, rows):
    colid = lax.broadcasted_iota(jnp.int32, (rows, 128), 1)
    rowid = lax.broadcasted_iota(jnp.int32, (rows, 128), 0)
    cs = h
    d = 1
    while d < 128:
        cs = cs + jnp.where(colid < 128 - d, pltpu.roll(cs, 128 - d, 1), 0)
        d *= 2
    rs = jnp.broadcast_to(jnp.sum(h, axis=1, keepdims=True), (rows, 128))
    d = 1
    while d < rows:
        rs = rs + jnp.where(rowid < rows - d, pltpu.roll(rs, rows - d, 0), 0)
        d *= 2
    rs_excl = jnp.where(rowid < rows - 1, pltpu.roll(rs, rows - 1, 0), 0)
    return cs + rs_excl


def _tc_sel1_body(h_ref, o_ref):
    h = jnp.sum(h_ref[...], axis=0)
    suf = _flat_suffix(h, 256)
    b1 = jnp.sum(jnp.where(suf >= _K, 1, 0)) - 1
    rowid = lax.broadcasted_iota(jnp.int32, (256, 128), 0)
    colid = lax.broadcasted_iota(jnp.int32, (256, 128), 1)
    beta = rowid * 128 + colid
    cnt_above = jnp.sum(jnp.where(beta > b1, h, 0))
    r = _K - cnt_above
    pos = (lax.broadcasted_iota(jnp.int32, (8, 128), 0) * 128
           + lax.broadcasted_iota(jnp.int32, (8, 128), 1))
    o_ref[...] = jnp.where(pos == 0, b1,
                           jnp.where(pos == 1, r,
                                     jnp.where(pos == 2, cnt_above, 0)))


_tc_sel1 = pl.pallas_call(
    _tc_sel1_body,
    out_shape=jax.ShapeDtypeStruct((8, 128), jnp.int32),
)


def _tc_sel2_body(h_ref, s1_ref, o_ref):
    h = jnp.sum(h_ref[...], axis=0)
    b1 = s1_ref[0, 0]
    r = s1_ref[0, 1]
    cnt_above = s1_ref[0, 2]
    suf = _flat_suffix(h, 512)
    m1 = jnp.sum(jnp.where(suf >= r, 1, 0)) - 1
    rowid = lax.broadcasted_iota(jnp.int32, (512, 128), 0)
    colid = lax.broadcasted_iota(jnp.int32, (512, 128), 1)
    beta = rowid * 128 + colid
    cnt_gt_low = jnp.sum(jnp.where(beta > m1, h, 0))
    cnt_gt = cnt_above + cnt_gt_low
    tbits = jnp.bitwise_or(lax.shift_left(b1, 16), m1)
    thr = lax.bitcast_convert_type(tbits, jnp.float32)
    prop = (5 * (2 * cnt_gt - _N)).astype(jnp.float32) / jnp.float32(_N)
    pos = (lax.broadcasted_iota(jnp.int32, (8, 128), 0) * 128
           + lax.broadcasted_iota(jnp.int32, (8, 128), 1))
    o_ref[...] = jnp.where(pos == 0, thr, jnp.where(pos == 1, prop, 0.0))


_tc_sel2 = pl.pallas_call(
    _tc_sel2_body,
    out_shape=jax.ShapeDtypeStruct((8, 128), jnp.float32),
)


_EWB = _N // 16


def _tc_ew_body(t_ref, p_ref, f_ref, oi_ref, om_ref):
    t = t_ref[0, 0]
    p = p_ref[...]
    f = f_ref[...]
    tv = f - p
    gt = jnp.abs(tv) > t
    mk = jnp.where(gt, jnp.float32(_BIAS), jnp.float32(-_BIAS))
    om_ref[...] = mk
    frac = 1.0 / (1.0 + jnp.exp(-mk))
    oi_ref[...] = p + tv * frac


_tc_ew = pl.pallas_call(
    _tc_ew_body,
    grid=(_N // _EWB,),
    in_specs=[
        pl.BlockSpec((8, 128), lambda i: (0, 0)),
        pl.BlockSpec((_EWB,), lambda i: (i,)),
        pl.BlockSpec((_EWB,), lambda i: (i,)),
    ],
    out_specs=[
        pl.BlockSpec((_EWB,), lambda i: (i,)),
        pl.BlockSpec((_EWB,), lambda i: (i,)),
    ],
    out_shape=[
        jax.ShapeDtypeStruct((_N,), jnp.float32),
        jax.ShapeDtypeStruct((_N,), jnp.float32),
    ],
)


def kernel(pretrained, finetuned):
    p = pretrained
    f = finetuned
    hist1 = _sc_hist_hi()(p, f)
    s1 = _tc_sel1(hist1)
    b1vec = jnp.broadcast_to(s1[0:1, 0], (_L,))
    hist2 = _sc_hist_lo()(p, f, b1vec)
    s2 = _tc_sel2(hist2, s1)
    interp, mask = _tc_ew(s2, p, f)
    return interp, mask, s2[0, 1]
